# baseline (speedup 1.0000x reference)
import functools

import numpy as np
import jax
import jax.numpy as jnp
from jax import lax
from jax.experimental import pallas as pl
from jax.experimental.pallas import tpu as pltpu
from jax.experimental.pallas import tpu_sc as plsc

POINTS = 4096
SB = 512
FB = 3584
CHUNK = 3136
F_SC = 7168

_ROT_A = (13, 15, 26, 6)
_ROT_B = (17, 29, 16, 24)


def _np_rotl(x, d):
    x = np.uint32(x) if np.isscalar(x) else x
    return ((x << np.uint32(d)) | (x >> np.uint32(32 - d))).astype(np.uint32)


def _np_threefry2x32(k1, k2, x0, x1):
    ks0, ks1 = np.uint32(k1), np.uint32(k2)
    ks2 = np.uint32(ks0 ^ ks1 ^ np.uint32(0x1BD11BDA))
    x0 = (x0 + ks0).astype(np.uint32)
    x1 = (x1 + ks1).astype(np.uint32)
    sched = ((ks1, ks2, 1), (ks2, ks0, 2), (ks0, ks1, 3),
             (ks1, ks2, 4), (ks2, ks0, 5))
    rots = (_ROT_A, _ROT_B, _ROT_A, _ROT_B, _ROT_A)
    for rot, (a, b, inc) in zip(rots, sched):
        for r in rot:
            x0 = (x0 + x1).astype(np.uint32)
            x1 = _np_rotl(x1, r)
            x1 = (x0 ^ x1).astype(np.uint32)
        x0 = (x0 + a).astype(np.uint32)
        x1 = (x1 + b + np.uint32(inc)).astype(np.uint32)
    return x0, x1


def _mesh_keys(batch):
    c1 = np.zeros(batch, np.uint32)
    c2 = np.arange(batch, dtype=np.uint32)
    b1, b2 = _np_threefry2x32(np.uint32(0), np.uint32(42), c1, c2)
    return np.stack([b1, b2], axis=1)


def _edges_sc(verts_t, faces_tp):
    batch, _, v_num = verts_t.shape
    f_pad = faces_tp.shape[2]
    half_f = f_pad // 2
    n_sub = half_f // CHUNK
    mesh = plsc.VectorSubcoreMesh(core_axis_name="c", subcore_axis_name="s")

    @functools.partial(
        pl.kernel, mesh=mesh,
        compiler_params=pltpu.CompilerParams(needs_layout_passes=False),
        out_type=jax.ShapeDtypeStruct((batch * 6 * f_pad,), jnp.float32),
        scratch_types=[
            pltpu.VMEM((v_num,), jnp.float32),
            pltpu.VMEM((CHUNK,), jnp.int32),
            pltpu.VMEM((CHUNK,), jnp.int32),
            pltpu.VMEM((CHUNK,), jnp.int32),
            pltpu.VMEM((CHUNK,), jnp.float32),
            pltpu.VMEM((CHUNK,), jnp.float32),
        ],
    )
    def body(verts_ref, faces_ref, out_ref, table, i0, i1, i2, e1, e2):
        wid = lax.axis_index("s") * 2 + lax.axis_index("c")
        b = wid // 2
        half = wid % 2
        for d in range(3):
            pltpu.sync_copy(verts_ref.at[pl.ds((b * 3 + d) * v_num, v_num)],
                            table)

            def sub_body(s, _):
                base = half * half_f + s * CHUNK
                for c, buf in ((0, i0), (1, i1), (2, i2)):
                    pltpu.sync_copy(
                        faces_ref.at[pl.ds((b * 3 + c) * f_pad + base, CHUNK)],
                        buf)

                @plsc.parallel_loop(0, CHUNK // 16)
                def gather_body(i):
                    sl = pl.ds(i * 16, 16)
                    v0 = plsc.load_gather(table, [i0[sl]])
                    v1 = plsc.load_gather(table, [i1[sl]])
                    v2 = plsc.load_gather(table, [i2[sl]])
                    e1[sl] = v1 - v0
                    e2[sl] = v2 - v0
                pltpu.sync_copy(
                    e1, out_ref.at[pl.ds((b * 6 + d) * f_pad + base, CHUNK)])
                pltpu.sync_copy(
                    e2,
                    out_ref.at[pl.ds((b * 6 + 3 + d) * f_pad + base, CHUNK)])
                return 0

            lax.fori_loop(0, n_sub, sub_body, 0)

    out = body(verts_t.reshape(-1), faces_tp.reshape(-1))
    return out.reshape(batch, 6, f_pad)


def _bits_sc(keys_flat, batch, f_real):
    rows_per_w = POINTS // 2
    mesh = plsc.VectorSubcoreMesh(core_axis_name="c", subcore_axis_name="s")

    @functools.partial(
        pl.kernel, mesh=mesh,
        compiler_params=pltpu.CompilerParams(needs_layout_passes=False),
        out_type=jax.ShapeDtypeStruct((batch * POINTS * F_SC,), jnp.int32),
        scratch_types=[
            pltpu.VMEM((2 * batch,), jnp.int32),
            pltpu.VMEM((F_SC,), jnp.int32),
        ],
    )
    def body(keys_ref, out_ref, ktab, row_buf):
        wid = lax.axis_index("s") * 2 + lax.axis_index("c")
        b = wid // 2
        half = wid % 2
        pltpu.sync_copy(keys_ref, ktab)
        splat = jnp.zeros((16,), jnp.int32)
        k1 = plsc.load_gather(ktab, [splat + 2 * b])
        k2 = plsc.load_gather(ktab, [splat + 2 * b + 1])
        ks2 = k1 ^ k2 ^ jnp.int32(0x1BD11BDA)
        lane = lax.iota(jnp.int32, 16)

        def row_body(s, _):
            s_glob = half * rows_per_w + s
            base = s_glob * f_real + k2

            @plsc.parallel_loop(0, F_SC // 16)
            def chunk(j):
                x1 = base + j * 16 + lane
                x0 = k1

                def rounds(x0, x1, rot):
                    for r in rot:
                        x0 = x0 + x1
                        x1 = (lax.shift_left(x1, r)
                              | lax.shift_right_logical(x1, 32 - r))
                        x1 = x0 ^ x1
                    return x0, x1

                x0, x1 = rounds(x0, x1, _ROT_A)
                x0 += k2
                x1 += ks2 + jnp.int32(1)
                x0, x1 = rounds(x0, x1, _ROT_B)
                x0 += ks2
                x1 += k1 + jnp.int32(2)
                x0, x1 = rounds(x0, x1, _ROT_A)
                x0 += k1
                x1 += k2 + jnp.int32(3)
                x0, x1 = rounds(x0, x1, _ROT_B)
                x0 += k2
                x1 += ks2 + jnp.int32(4)
                x0, x1 = rounds(x0, x1, _ROT_A)
                x0 += ks2
                x1 += k1 + jnp.int32(5)
                row_buf[pl.ds(j * 16, 16)] = x0 ^ x1

            pltpu.sync_copy(
                row_buf,
                out_ref.at[pl.ds((b * POINTS + s_glob) * F_SC, F_SC)])
            return 0

        lax.fori_loop(0, rows_per_w, row_body, 0)

    return body(keys_flat)


def _tf_rounds(x0, x1, rot):
    for r in rot:
        x0 = x0 + x1
        x1 = (x1 << jnp.uint32(r)) | (x1 >> jnp.uint32(32 - r))
        x1 = x0 ^ x1
    return x0, x1


def _ninv_w(e_ref, f0, f_real):
    e = e_ref[0]
    d1x, d1y, d1z = e[0:1], e[1:2], e[2:3]
    d2x, d2y, d2z = e[3:4], e[4:5], e[5:6]
    cx = d1y * d2z - d1z * d2y
    cy = d1z * d2x - d1x * d2z
    cz = d1x * d2y - d1y * d2x
    q = cx * cx + cy * cy + cz * cz
    f_ids = f0 + lax.broadcasted_iota(jnp.int32, (1, FB), 1)
    return f_ids, jnp.where(f_ids < f_real,
                            -1.0 / (jnp.sqrt(q) * 0.5 + 1e-12), -jnp.inf)


def _race_update(v, f_ids, fb, last_fb, bv_ref, bi_ref):
    bm = jnp.min(v, axis=1, keepdims=True)
    cand = jnp.where(v == bm, f_ids, jnp.int32(0x7FFFFFFF))
    bi = jnp.min(cand, axis=1, keepdims=True)

    @pl.when(fb == 0)
    def _():
        bv_ref[...] = bm
        bi_ref[...] = bi

    @pl.when(fb > 0)
    def _():
        upd = bm < bv_ref[...]
        bv_ref[...] = jnp.where(upd, bm, bv_ref[...])
        bi_ref[...] = jnp.where(upd, bi, bi_ref[...])


def _main_body(f_real, n_fblocks, keys_ref, e_ref, bv_out, bi_out,
               bv_ref, bi_ref):
    b = pl.program_id(0)
    sb = pl.program_id(1)
    fb = pl.program_id(2)
    k1 = keys_ref[b, 0]
    k2 = keys_ref[b, 1]
    ks2 = k1 ^ k2 ^ jnp.uint32(0x1BD11BDA)

    s0 = sb * SB
    f0 = F_SC + fb * FB
    f_ids, ninv_w = _ninv_w(e_ref, f0, f_real)

    row = (jnp.uint32(f_real)
           * lax.broadcasted_iota(jnp.uint32, (SB, 1), 0))
    col = (jnp.uint32(s0 * f_real + f0) + k2
           + lax.broadcasted_iota(jnp.uint32, (1, FB), 1))
    x1 = row + col
    x0 = jnp.full((SB, FB), k1, jnp.uint32)
    x0, x1 = _tf_rounds(x0, x1, _ROT_A)
    x0 += k2
    x1 += ks2 + jnp.uint32(1)
    x0, x1 = _tf_rounds(x0, x1, _ROT_B)
    x0 += ks2
    x1 += k1 + jnp.uint32(2)
    x0, x1 = _tf_rounds(x0, x1, _ROT_A)
    x0 += k1
    x1 += k2 + jnp.uint32(3)
    x0, x1 = _tf_rounds(x0, x1, _ROT_B)
    x0 += k2
    x1 += ks2 + jnp.uint32(4)
    x0, x1 = _tf_rounds(x0, x1, _ROT_A)
    x0 += ks2
    x1 += k1 + jnp.uint32(5)
    bits = x0 ^ x1

    tiny = jnp.float32(np.finfo(np.float32).tiny)
    fbits = (bits >> jnp.uint32(9)) | jnp.uint32(0x3F800000)
    u = (lax.bitcast_convert_type(fbits, jnp.float32) - 1.0) + tiny
    v = jnp.log(u) * ninv_w

    _race_update(v, f_ids, fb, n_fblocks - 1, bv_ref, bi_ref)

    @pl.when(fb == n_fblocks - 1)
    def _():
        bv_out[0] = bv_ref[...]
        bi_out[0] = bi_ref[...]


def _merge_body(f_real, n_fblocks, bits_ref, e_ref, bv1_ref, bi1_ref,
                out_ref, bv_ref, bi_ref):
    fb = pl.program_id(2)
    f0 = fb * FB
    f_ids, ninv_w = _ninv_w(e_ref, f0, f_real)

    bits = bits_ref[0]
    tiny = jnp.float32(np.finfo(np.float32).tiny)
    fbits = ((lax.bitcast_convert_type(bits, jnp.uint32) >> jnp.uint32(9))
             | jnp.uint32(0x3F800000))
    u = (lax.bitcast_convert_type(fbits, jnp.float32) - 1.0) + tiny
    v = jnp.log(u) * ninv_w

    _race_update(v, f_ids, fb, n_fblocks - 1, bv_ref, bi_ref)

    @pl.when(fb == n_fblocks - 1)
    def _():
        upd = bv_ref[...] <= bv1_ref[0]
        out_ref[0] = jnp.where(upd, bi_ref[...], bi1_ref[0])


def _sample_faces(edges, bits, keys, f_real):
    batch, _, f_pad = edges.shape
    n_main = (f_pad - F_SC) // FB
    n_sblocks = POINTS // SB
    bv1, bi1 = pl.pallas_call(
        functools.partial(_main_body, f_real, n_main),
        grid=(batch, n_sblocks, n_main),
        in_specs=[
            pl.BlockSpec(memory_space=pltpu.SMEM),
            pl.BlockSpec((1, 6, FB), lambda b, sb, fb: (b, 0, fb + F_SC // FB)),
        ],
        out_specs=[
            pl.BlockSpec((1, SB, 1), lambda b, sb, fb: (b, sb, 0)),
            pl.BlockSpec((1, SB, 1), lambda b, sb, fb: (b, sb, 0)),
        ],
        out_shape=[
            jax.ShapeDtypeStruct((batch, POINTS, 1), jnp.float32),
            jax.ShapeDtypeStruct((batch, POINTS, 1), jnp.int32),
        ],
        scratch_shapes=[
            pltpu.VMEM((SB, 1), jnp.float32),
            pltpu.VMEM((SB, 1), jnp.int32),
        ],
        compiler_params=pltpu.CompilerParams(
            dimension_semantics=("parallel", "parallel", "arbitrary")),
    )(keys, edges)

    n_sc = F_SC // FB
    out = pl.pallas_call(
        functools.partial(_merge_body, f_real, n_sc),
        grid=(batch, n_sblocks, n_sc),
        in_specs=[
            pl.BlockSpec((1, SB, FB), lambda b, sb, fb: (b, sb, fb)),
            pl.BlockSpec((1, 6, FB), lambda b, sb, fb: (b, 0, fb)),
            pl.BlockSpec((1, SB, 1), lambda b, sb, fb: (b, sb, 0)),
            pl.BlockSpec((1, SB, 1), lambda b, sb, fb: (b, sb, 0)),
        ],
        out_specs=pl.BlockSpec((1, SB, 1), lambda b, sb, fb: (b, sb, 0)),
        out_shape=jax.ShapeDtypeStruct((batch, POINTS, 1), jnp.int32),
        scratch_shapes=[
            pltpu.VMEM((SB, 1), jnp.float32),
            pltpu.VMEM((SB, 1), jnp.int32),
        ],
        compiler_params=pltpu.CompilerParams(
            dimension_semantics=("parallel", "parallel", "arbitrary")),
    )(bits, edges, bv1, bi1)
    return out.reshape(batch, POINTS)


def kernel(vertices_batch, faces_batch):
    batch, f_real = faces_batch.shape[0], faces_batch.shape[1]
    f_pad = ((f_real + FB - 1) // FB) * FB
    faces = faces_batch.astype(jnp.int32)
    faces_tp = jnp.transpose(
        jnp.pad(faces, ((0, 0), (0, f_pad - f_real), (0, 0))), (0, 2, 1))
    verts_t = jnp.transpose(vertices_batch, (0, 2, 1))
    keys = jnp.asarray(_mesh_keys(batch))
    keys_flat = jnp.asarray(
        _mesh_keys(batch).reshape(-1).view(np.int32))
    edges = _edges_sc(verts_t, faces_tp)
    bits = _bits_sc(keys_flat, batch, f_real)
    bits = bits.reshape(batch, POINTS, F_SC)
    return _sample_faces(edges, bits, keys, f_real)


# speedup vs baseline: 1.1406x; 1.1406x over previous
import functools

import numpy as np
import jax
import jax.numpy as jnp
from jax import lax
from jax.experimental import pallas as pl
from jax.experimental.pallas import tpu as pltpu
from jax.experimental.pallas import tpu_sc as plsc

POINTS = 4096
SB = 512
FB = 3584
CHUNK = 3136

_ROT_A = (13, 15, 26, 6)
_ROT_B = (17, 29, 16, 24)


def _np_rotl(x, d):
    x = np.uint32(x) if np.isscalar(x) else x
    return ((x << np.uint32(d)) | (x >> np.uint32(32 - d))).astype(np.uint32)


def _np_threefry2x32(k1, k2, x0, x1):
    ks0, ks1 = np.uint32(k1), np.uint32(k2)
    ks2 = np.uint32(ks0 ^ ks1 ^ np.uint32(0x1BD11BDA))
    x0 = (x0 + ks0).astype(np.uint32)
    x1 = (x1 + ks1).astype(np.uint32)
    sched = ((ks1, ks2, 1), (ks2, ks0, 2), (ks0, ks1, 3),
             (ks1, ks2, 4), (ks2, ks0, 5))
    rots = (_ROT_A, _ROT_B, _ROT_A, _ROT_B, _ROT_A)
    for rot, (a, b, inc) in zip(rots, sched):
        for r in rot:
            x0 = (x0 + x1).astype(np.uint32)
            x1 = _np_rotl(x1, r)
            x1 = (x0 ^ x1).astype(np.uint32)
        x0 = (x0 + a).astype(np.uint32)
        x1 = (x1 + b + np.uint32(inc)).astype(np.uint32)
    return x0, x1


def _mesh_keys(batch):
    c1 = np.zeros(batch, np.uint32)
    c2 = np.arange(batch, dtype=np.uint32)
    b1, b2 = _np_threefry2x32(np.uint32(0), np.uint32(42), c1, c2)
    return np.stack([b1, b2], axis=1)


def _edges_sc(verts_t, faces_tp):
    batch, _, v_num = verts_t.shape
    f_pad = faces_tp.shape[2]
    half_f = f_pad // 2
    n_sub = half_f // CHUNK
    mesh = plsc.VectorSubcoreMesh(core_axis_name="c", subcore_axis_name="s")

    @functools.partial(
        pl.kernel, mesh=mesh,
        compiler_params=pltpu.CompilerParams(needs_layout_passes=False),
        out_type=jax.ShapeDtypeStruct((batch * 6 * f_pad,), jnp.float32),
        scratch_types=[
            pltpu.VMEM((v_num,), jnp.float32),
            pltpu.VMEM((CHUNK,), jnp.int32),
            pltpu.VMEM((CHUNK,), jnp.int32),
            pltpu.VMEM((CHUNK,), jnp.int32),
            pltpu.VMEM((CHUNK,), jnp.float32),
            pltpu.VMEM((CHUNK,), jnp.float32),
        ],
    )
    def body(verts_ref, faces_ref, out_ref, table, i0, i1, i2, e1, e2):
        wid = lax.axis_index("s") * 2 + lax.axis_index("c")
        b = wid // 2
        half = wid % 2
        for d in range(3):
            pltpu.sync_copy(verts_ref.at[pl.ds((b * 3 + d) * v_num, v_num)],
                            table)

            def sub_body(s, _):
                base = half * half_f + s * CHUNK
                for c, buf in ((0, i0), (1, i1), (2, i2)):
                    pltpu.sync_copy(
                        faces_ref.at[pl.ds((b * 3 + c) * f_pad + base, CHUNK)],
                        buf)

                @plsc.parallel_loop(0, CHUNK // 16)
                def gather_body(i):
                    sl = pl.ds(i * 16, 16)
                    v0 = plsc.load_gather(table, [i0[sl]])
                    v1 = plsc.load_gather(table, [i1[sl]])
                    v2 = plsc.load_gather(table, [i2[sl]])
                    e1[sl] = v1 - v0
                    e2[sl] = v2 - v0
                pltpu.sync_copy(
                    e1, out_ref.at[pl.ds((b * 6 + d) * f_pad + base, CHUNK)])
                pltpu.sync_copy(
                    e2,
                    out_ref.at[pl.ds((b * 6 + 3 + d) * f_pad + base, CHUNK)])
                return 0

            lax.fori_loop(0, n_sub, sub_body, 0)

    out = body(verts_t.reshape(-1), faces_tp.reshape(-1))
    return out.reshape(batch, 6, f_pad)


def _tf_rounds(x0, x1, rot):
    for r in rot:
        x0 = x0 + x1
        x1 = (x1 << jnp.uint32(r)) | (x1 >> jnp.uint32(32 - r))
        x1 = x0 ^ x1
    return x0, x1


def _sampler_body(f_real, n_fblocks, keys_ref, e_ref, out_ref, bv_ref, bi_ref):
    b = pl.program_id(0)
    sb = pl.program_id(1)
    fb = pl.program_id(2)
    k1 = keys_ref[b, 0]
    k2 = keys_ref[b, 1]
    ks2 = k1 ^ k2 ^ jnp.uint32(0x1BD11BDA)

    s0 = sb * SB
    f0 = fb * FB

    e = e_ref[0]
    d1x, d1y, d1z = e[0:1], e[1:2], e[2:3]
    d2x, d2y, d2z = e[3:4], e[4:5], e[5:6]
    cx = d1y * d2z - d1z * d2y
    cy = d1z * d2x - d1x * d2z
    cz = d1x * d2y - d1y * d2x
    q = cx * cx + cy * cy + cz * cz
    f_ids = f0 + lax.broadcasted_iota(jnp.int32, (1, FB), 1)
    ninv_w = jnp.where(f_ids < f_real,
                       -1.0 / (jnp.sqrt(q) * 0.5 + 1e-12), -jnp.inf)

    row = (jnp.uint32(f_real)
           * lax.broadcasted_iota(jnp.uint32, (SB, 1), 0))
    col = (jnp.uint32(s0 * f_real + f0) + k2
           + lax.broadcasted_iota(jnp.uint32, (1, FB), 1))
    x1 = row + col
    x0 = jnp.full((SB, FB), k1, jnp.uint32)
    x0, x1 = _tf_rounds(x0, x1, _ROT_A)
    x0 += k2
    x1 += ks2 + jnp.uint32(1)
    x0, x1 = _tf_rounds(x0, x1, _ROT_B)
    x0 += ks2
    x1 += k1 + jnp.uint32(2)
    x0, x1 = _tf_rounds(x0, x1, _ROT_A)
    x0 += k1
    x1 += k2 + jnp.uint32(3)
    x0, x1 = _tf_rounds(x0, x1, _ROT_B)
    x0 += k2
    x1 += ks2 + jnp.uint32(4)
    x0, x1 = _tf_rounds(x0, x1, _ROT_A)
    x0 += ks2
    x1 += k1 + jnp.uint32(5)
    bits = x0 ^ x1

    tiny = jnp.float32(np.finfo(np.float32).tiny)
    fbits = (bits >> jnp.uint32(9)) | jnp.uint32(0x3F800000)
    u = (lax.bitcast_convert_type(fbits, jnp.float32) - 1.0) + tiny
    v = jnp.log(u) * ninv_w

    bm = jnp.min(v, axis=1, keepdims=True)
    cand = jnp.where(v == bm, f_ids, jnp.int32(0x7FFFFFFF))
    bi = jnp.min(cand, axis=1, keepdims=True)

    @pl.when(fb == 0)
    def _():
        bv_ref[...] = bm
        bi_ref[...] = bi

    @pl.when(fb > 0)
    def _():
        upd = bm < bv_ref[...]
        bv_ref[...] = jnp.where(upd, bm, bv_ref[...])
        bi_ref[...] = jnp.where(upd, bi, bi_ref[...])

    @pl.when(fb == n_fblocks - 1)
    def _():
        out_ref[0] = bi_ref[...]


def _sample_faces(edges, keys, f_real):
    batch, _, f_pad = edges.shape
    n_fblocks = f_pad // FB
    n_sblocks = POINTS // SB
    body = functools.partial(_sampler_body, f_real, n_fblocks)
    out = pl.pallas_call(
        body,
        grid=(batch, n_sblocks, n_fblocks),
        in_specs=[
            pl.BlockSpec(memory_space=pltpu.SMEM),
            pl.BlockSpec((1, 6, FB), lambda b, sb, fb: (b, 0, fb)),
        ],
        out_specs=pl.BlockSpec((1, SB, 1), lambda b, sb, fb: (b, sb, 0)),
        out_shape=jax.ShapeDtypeStruct((batch, POINTS, 1), jnp.int32),
        scratch_shapes=[
            pltpu.VMEM((SB, 1), jnp.float32),
            pltpu.VMEM((SB, 1), jnp.int32),
        ],
        compiler_params=pltpu.CompilerParams(
            dimension_semantics=("parallel", "parallel", "arbitrary")),
    )(keys, edges)
    return out.reshape(batch, POINTS)


def kernel(vertices_batch, faces_batch):
    batch, f_real = faces_batch.shape[0], faces_batch.shape[1]
    f_pad = ((f_real + FB - 1) // FB) * FB
    faces = faces_batch.astype(jnp.int32)
    faces_tp = jnp.transpose(
        jnp.pad(faces, ((0, 0), (0, f_pad - f_real), (0, 0))), (0, 2, 1))
    verts_t = jnp.transpose(vertices_batch, (0, 2, 1))
    edges = _edges_sc(verts_t, faces_tp)
    keys = jnp.asarray(_mesh_keys(batch))
    return _sample_faces(edges, keys, f_real)
